# baseline (speedup 1.0000x reference)
import functools

import jax
import jax.numpy as jnp
from jax import lax
from jax.experimental import pallas as pl
from jax.experimental.pallas import tpu as pltpu
from jax.experimental.pallas import tpu_sc as plsc

N = 16384
D = 32
K = 8192
BLK = 512
NBLK = N // BLK
COMMITMENT_COST = 0.25


def _dist_argmin_body(x_ref, cb_ref, idx_ref, minsq_ref):
    xb = x_ref[...]
    cb = cb_ref[...]
    c2 = jnp.sum(cb * cb, axis=1)
    x2 = jnp.sum(xb * xb, axis=1)
    d = lax.dot_general(
        xb.astype(jnp.bfloat16), cb.astype(jnp.bfloat16),
        (((1,), (1,)), ((), ())),
        preferred_element_type=jnp.float32,
    )
    sq = (x2[:, None] + c2[None, :]) - 2.0 * d
    dist = jnp.sqrt(jnp.maximum(sq, 0.0))
    H = K // 2
    d0 = dist[:, :H]
    d1 = dist[:, H:]
    iota = lax.broadcasted_iota(jnp.int32, (BLK, H), 1)
    m0 = jnp.min(d0, axis=1)
    i0 = jnp.min(jnp.where(d0 == m0[:, None], iota, H), axis=1)
    m1 = jnp.min(d1, axis=1)
    i1 = jnp.min(jnp.where(d1 == m1[:, None], iota, H), axis=1)
    m0r = m0.astype(jnp.bfloat16).astype(jnp.float32)
    use1 = m1 < m0r
    idx = jnp.where(use1, i1 + H, i0)
    m = jnp.where(use1, m1, m0)
    idx_ref[...] = idx.reshape(1, 1, BLK)
    minsq_ref[...] = (m * m).reshape(1, 1, BLK)


_dist_argmin = pl.pallas_call(
    _dist_argmin_body,
    grid=(NBLK,),
    compiler_params=pltpu.CompilerParams(dimension_semantics=("parallel",)),
    in_specs=[
        pl.BlockSpec((BLK, D), lambda i: (i, 0)),
        pl.BlockSpec((K, D), lambda i: (0, 0)),
    ],
    out_specs=[
        pl.BlockSpec((1, 1, BLK), lambda i: (i, 0, 0)),
        pl.BlockSpec((1, 1, BLK), lambda i: (i, 0, 0)),
    ],
    out_shape=[
        jax.ShapeDtypeStruct((NBLK, 1, BLK), jnp.int32),
        jax.ShapeDtypeStruct((NBLK, 1, BLK), jnp.float32),
    ],
)


_NC = 2
_NS = 16
_NW = _NC * _NS
_BPW = N // _NW

@functools.cache
def _make_sc_gather():
    mesh = plsc.VectorSubcoreMesh(core_axis_name="c", subcore_axis_name="s")

    @functools.partial(
        pl.kernel,
        mesh=mesh,
        compiler_params=pltpu.CompilerParams(use_tc_tiling_on_sc=False),
        out_type=jax.ShapeDtypeStruct((N, D), jnp.float32),
        scratch_types=[
            pltpu.VMEM((_BPW,), jnp.int32),
            pltpu.VMEM((_BPW, D), jnp.float32),
            pltpu.SemaphoreType.DMA,
        ],
    )
    def _sc_gather(table_hbm, idx_hbm, out_hbm, idx_v, rows_v, sem):
        wid = lax.axis_index("s") * _NC + lax.axis_index("c")
        base = wid * _BPW
        pltpu.sync_copy(idx_hbm.at[pl.ds(base, _BPW)], idx_v)
        pltpu.async_copy(table_hbm.at[idx_v], rows_v, sem).wait()
        pltpu.sync_copy(rows_v, out_hbm.at[pl.ds(base, _BPW)])

    return _sc_gather


def kernel(inputs, codebook):
    flat = inputs.reshape(N, D)
    idx3, minsq3 = _dist_argmin(flat, codebook)
    idx = idx3.reshape(N)
    quantized = _make_sc_gather()(codebook, idx)
    loss = ((1.0 + COMMITMENT_COST) / (N * D)) * jnp.sum(minsq3)
    return quantized.reshape(inputs.shape), loss, idx


# speedup vs baseline: 1.2860x; 1.2860x over previous
import functools

import jax
import jax.numpy as jnp
from jax import lax
from jax.experimental import pallas as pl
from jax.experimental.pallas import tpu as pltpu
from jax.experimental.pallas import tpu_sc as plsc

N = 16384
D = 32
K = 8192
BLK = 512
NBLK = N // BLK
COMMITMENT_COST = 0.25


def _dist_argmin_body(x_ref, cb_ref, idx_ref, minsq_ref):
    xb = x_ref[...]
    cb = cb_ref[...]
    c2 = jnp.sum(cb * cb, axis=1)
    x2 = jnp.sum(xb * xb, axis=1)
    d = lax.dot_general(
        xb.astype(jnp.bfloat16), cb.astype(jnp.bfloat16),
        (((1,), (1,)), ((), ())),
        preferred_element_type=jnp.float32,
    )
    sq = (x2[:, None] + c2[None, :]) - 2.0 * d
    H = K // 2
    msq0 = jnp.min(sq[:, :H], axis=1)
    msq1 = jnp.min(sq[:, H:], axis=1)
    s0 = jnp.sqrt(jnp.maximum(msq0, 0.0))
    s1 = jnp.sqrt(jnp.maximum(msq1, 0.0))

    def class_hi(s, msq):
        c0 = s * s
        cb_ = lax.bitcast_convert_type(c0, jnp.int32)
        best = jnp.maximum(c0, msq)
        for k in (-2, -1, 1, 2, 3):
            cand = lax.bitcast_convert_type(cb_ + k, jnp.float32)
            ok = jnp.sqrt(jnp.maximum(cand, 0.0)) == s
            best = jnp.where(ok, jnp.maximum(best, cand), best)
        return best

    hi0 = class_hi(s0, msq0)
    hi1 = class_hi(s1, msq1)
    iota = lax.broadcasted_iota(jnp.int32, (BLK, H), 1)
    i0 = jnp.min(jnp.where(sq[:, :H] <= hi0[:, None], iota, H), axis=1)
    i1 = jnp.min(jnp.where(sq[:, H:] <= hi1[:, None], iota, H), axis=1)
    m0r = s0.astype(jnp.bfloat16).astype(jnp.float32)
    use1 = s1 < m0r
    idx = jnp.where(use1, i1 + H, i0)
    m = jnp.where(use1, s1, s0)
    idx_ref[...] = idx.reshape(1, 1, BLK)
    minsq_ref[...] = (m * m).reshape(1, 1, BLK)


_dist_argmin = pl.pallas_call(
    _dist_argmin_body,
    grid=(NBLK,),
    in_specs=[
        pl.BlockSpec((BLK, D), lambda i: (i, 0)),
        pl.BlockSpec((K, D), lambda i: (0, 0)),
    ],
    out_specs=[
        pl.BlockSpec((1, 1, BLK), lambda i: (i, 0, 0)),
        pl.BlockSpec((1, 1, BLK), lambda i: (i, 0, 0)),
    ],
    out_shape=[
        jax.ShapeDtypeStruct((NBLK, 1, BLK), jnp.int32),
        jax.ShapeDtypeStruct((NBLK, 1, BLK), jnp.float32),
    ],
)


_NC = 2
_NS = 16
_NW = _NC * _NS
_BPW = N // _NW

@functools.cache
def _make_sc_gather():
    mesh = plsc.VectorSubcoreMesh(core_axis_name="c", subcore_axis_name="s")

    @functools.partial(
        pl.kernel,
        mesh=mesh,
        compiler_params=pltpu.CompilerParams(use_tc_tiling_on_sc=False),
        out_type=jax.ShapeDtypeStruct((N, D), jnp.float32),
        scratch_types=[
            pltpu.VMEM((_BPW,), jnp.int32),
            pltpu.VMEM((_BPW, D), jnp.float32),
            pltpu.SemaphoreType.DMA,
        ],
    )
    def _sc_gather(table_hbm, idx_hbm, out_hbm, idx_v, rows_v, sem):
        wid = lax.axis_index("s") * _NC + lax.axis_index("c")
        base = wid * _BPW
        pltpu.sync_copy(idx_hbm.at[pl.ds(base, _BPW)], idx_v)
        pltpu.async_copy(table_hbm.at[idx_v], rows_v, sem).wait()
        pltpu.sync_copy(rows_v, out_hbm.at[pl.ds(base, _BPW)])

    return _sc_gather


def kernel(inputs, codebook):
    flat = inputs.reshape(N, D)
    idx3, minsq3 = _dist_argmin(flat, codebook)
    idx = idx3.reshape(N)
    quantized = _make_sc_gather()(codebook, idx)
    loss = ((1.0 + COMMITMENT_COST) / (N * D)) * jnp.sum(minsq3)
    return quantized.reshape(inputs.shape), loss, idx


# speedup vs baseline: 1.3660x; 1.0622x over previous
import functools

import jax
import jax.numpy as jnp
from jax import lax
from jax.experimental import pallas as pl
from jax.experimental.pallas import tpu as pltpu
from jax.experimental.pallas import tpu_sc as plsc

N = 16384
D = 32
K = 8192
BLK = 1024
NBLK = N // BLK
COMMITMENT_COST = 0.25


def _dist_argmin_body(x_ref, cb_ref, idx_ref, minsq_ref):
    xb = x_ref[...]
    cb = cb_ref[...]
    c2 = jnp.sum(cb * cb, axis=1)
    x2 = jnp.sum(xb * xb, axis=1)
    d = lax.dot_general(
        xb.astype(jnp.bfloat16), cb.astype(jnp.bfloat16),
        (((1,), (1,)), ((), ())),
        preferred_element_type=jnp.float32,
    )
    sq = (x2[:, None] + c2[None, :]) - 2.0 * d
    H = K // 2
    msq0 = jnp.min(sq[:, :H], axis=1)
    msq1 = jnp.min(sq[:, H:], axis=1)
    s0 = jnp.sqrt(jnp.maximum(msq0, 0.0))
    s1 = jnp.sqrt(jnp.maximum(msq1, 0.0))

    def class_hi(s, msq):
        c0 = s * s
        cb_ = lax.bitcast_convert_type(c0, jnp.int32)
        best = jnp.maximum(c0, msq)
        for k in (-2, -1, 1, 2, 3):
            cand = lax.bitcast_convert_type(cb_ + k, jnp.float32)
            ok = jnp.sqrt(jnp.maximum(cand, 0.0)) == s
            best = jnp.where(ok, jnp.maximum(best, cand), best)
        return best

    hi0 = class_hi(s0, msq0)
    hi1 = class_hi(s1, msq1)
    iota = lax.broadcasted_iota(jnp.int32, (BLK, H), 1)
    i0 = jnp.min(jnp.where(sq[:, :H] <= hi0[:, None], iota, H), axis=1)
    i1 = jnp.min(jnp.where(sq[:, H:] <= hi1[:, None], iota, H), axis=1)
    m0r = s0.astype(jnp.bfloat16).astype(jnp.float32)
    use1 = s1 < m0r
    idx = jnp.where(use1, i1 + H, i0)
    m = jnp.where(use1, s1, s0)
    idx_ref[...] = idx.reshape(1, 1, BLK)
    minsq_ref[...] = (m * m).reshape(1, 1, BLK)


_dist_argmin = pl.pallas_call(
    _dist_argmin_body,
    grid=(NBLK,),
    in_specs=[
        pl.BlockSpec((BLK, D), lambda i: (i, 0)),
        pl.BlockSpec((K, D), lambda i: (0, 0)),
    ],
    out_specs=[
        pl.BlockSpec((1, 1, BLK), lambda i: (i, 0, 0)),
        pl.BlockSpec((1, 1, BLK), lambda i: (i, 0, 0)),
    ],
    out_shape=[
        jax.ShapeDtypeStruct((NBLK, 1, BLK), jnp.int32),
        jax.ShapeDtypeStruct((NBLK, 1, BLK), jnp.float32),
    ],
)


_NC = 2
_NS = 16
_NW = _NC * _NS
_BPW = N // _NW

@functools.cache
def _make_sc_gather():
    mesh = plsc.VectorSubcoreMesh(core_axis_name="c", subcore_axis_name="s")

    @functools.partial(
        pl.kernel,
        mesh=mesh,
        compiler_params=pltpu.CompilerParams(use_tc_tiling_on_sc=False),
        out_type=jax.ShapeDtypeStruct((N, D), jnp.float32),
        scratch_types=[
            pltpu.VMEM((_BPW,), jnp.int32),
            pltpu.VMEM((_BPW, D), jnp.float32),
            pltpu.SemaphoreType.DMA,
        ],
    )
    def _sc_gather(table_hbm, idx_hbm, out_hbm, idx_v, rows_v, sem):
        wid = lax.axis_index("s") * _NC + lax.axis_index("c")
        base = wid * _BPW
        pltpu.sync_copy(idx_hbm.at[pl.ds(base, _BPW)], idx_v)
        pltpu.async_copy(table_hbm.at[idx_v], rows_v, sem).wait()
        pltpu.sync_copy(rows_v, out_hbm.at[pl.ds(base, _BPW)])

    return _sc_gather


def kernel(inputs, codebook):
    flat = inputs.reshape(N, D)
    idx3, minsq3 = _dist_argmin(flat, codebook)
    idx = idx3.reshape(N)
    quantized = _make_sc_gather()(codebook, idx)
    loss = ((1.0 + COMMITMENT_COST) / (N * D)) * jnp.sum(minsq3)
    return quantized.reshape(inputs.shape), loss, idx
